# baseline (speedup 1.0000x reference)
import functools

import jax
import jax.numpy as jnp
from jax import lax
from jax.experimental import pallas as pl
from jax.experimental.pallas import tpu as pltpu
from jax.experimental.pallas import tpu_sc as plsc

NC = 2
NS = 16
L = 16
NW = NC * NS

EMBED_DIM = 32
MARGIN = 1.0

QBITS = 18
Q = 1 << QBITS
PACK_BLK = 512


def _repack_body(c0, c1, c2, c3, out):
    for c, ref in enumerate((c0, c1, c2, c3)):
        out[:, c * EMBED_DIM:(c + 1) * EMBED_DIM] = ref[...].T


def _pack_table(table_t):
    nsteps = Q // PACK_BLK
    vblocks = table_t.shape[1] // PACK_BLK
    in_specs = [
        pl.BlockSpec(
            (EMBED_DIM, PACK_BLK),
            functools.partial(
                lambda c, i: (0, jnp.minimum(c * nsteps + i, vblocks)), c
            ),
        )
        for c in range(4)
    ]
    return pl.pallas_call(
        _repack_body,
        grid=(nsteps,),
        in_specs=in_specs,
        out_specs=pl.BlockSpec((PACK_BLK, 4 * EMBED_DIM), lambda i: (i, 0)),
        out_shape=jax.ShapeDtypeStruct((Q, 4 * EMBED_DIM), jnp.float32),
    )(table_t, table_t, table_t, table_t)


def _rsqrt(x):
    i = plsc.bitcast(x, jnp.int32)
    i = jnp.int32(0x5F3759DF) - (i >> 1)
    y = plsc.bitcast(i, jnp.float32)
    for _ in range(3):
        y = y * (jnp.float32(1.5) - (jnp.float32(0.5) * x * y) * y)
    return y


def _make_sc_kernel(batch):
    assert batch % (8 * NW) == 0
    bpw = batch // NW
    chunk = 256
    nchunks = bpw // chunk
    groups = chunk // L

    mesh = plsc.VectorSubcoreMesh(
        core_axis_name="c", subcore_axis_name="s", num_cores=NC, num_subcores=NS
    )

    @functools.partial(
        pl.kernel,
        out_type=jax.ShapeDtypeStruct((1, batch), jnp.float32),
        mesh=mesh,
        scratch_types=[
            pltpu.VMEM((6, bpw), jnp.int32),
            pltpu.VMEM((chunk,), jnp.int32),
            pltpu.VMEM((chunk,), jnp.int32),
            pltpu.VMEM((chunk,), jnp.int32),
            pltpu.VMEM((chunk, 4 * EMBED_DIM), jnp.float32),
            pltpu.VMEM((chunk, 4 * EMBED_DIM), jnp.float32),
            pltpu.VMEM((chunk, 4 * EMBED_DIM), jnp.float32),
            pltpu.VMEM((bpw,), jnp.float32),
            pltpu.VMEM((bpw,), jnp.float32),
            pltpu.SemaphoreType.DMA,
        ],
        compiler_params=pltpu.CompilerParams(
            needs_layout_passes=False, use_tc_tiling_on_sc=False
        ),
    )
    def sc_kernel(
        hp_hbm,
        lp_hbm,
        tp_hbm,
        hn_hbm,
        ln_hbm,
        tn_hbm,
        ent_hbm,
        lab_hbm,
        out_hbm,
        row_v,
        hi_v,
        li_v,
        ti_v,
        h_v,
        l_v,
        t_v,
        dp_v,
        out_v,
        sem,
    ):
        wid = lax.axis_index("s") * NC + lax.axis_index("c")
        base = wid * bpw

        for k, src in enumerate((hp_hbm, lp_hbm, tp_hbm, hn_hbm, ln_hbm, tn_hbm)):
            pltpu.sync_copy(src.at[pl.ds(base, bpw)], row_v.at[k])

        def distance(rid, cbase_h, cbase_l, cbase_t):
            z = jnp.zeros((L,), jnp.float32)
            hh = tt = ll = hl = ht = lt = z
            for j in range(EMBED_DIM):
                h = plsc.load_gather(h_v, [rid, cbase_h + j])
                l = plsc.load_gather(l_v, [rid, cbase_l + j])
                t = plsc.load_gather(t_v, [rid, cbase_t + j])
                hh = hh + h * h
                tt = tt + t * t
                ll = ll + l * l
                hl = hl + h * l
                ht = ht + h * t
                lt = lt + l * t
            a = _rsqrt(hh)
            b = _rsqrt(tt)
            two = jnp.float32(2.0)
            dsq = two + ll + two * a * hl - two * (a * b) * ht - two * b * lt
            dsq = jnp.maximum(dsq, jnp.float32(0.0))
            return dsq * _rsqrt(dsq)

        lid = lax.iota(jnp.int32, L)

        def phase(kh, kl, kt, is_pos):
            def do_chunk(ck, _):
                off = ck * chunk

                def stage(v, _):
                    svl = pl.ds(off + v * L, L)
                    dvl = pl.ds(v * L, L)
                    m = jnp.int32(Q - 1)
                    hi_v[dvl] = lax.bitwise_and(row_v[kh, svl], m)
                    li_v[dvl] = lax.bitwise_and(row_v[kl, svl], m)
                    ti_v[dvl] = lax.bitwise_and(row_v[kt, svl], m)
                    return _

                lax.fori_loop(0, groups, stage, 0)
                cph = pltpu.async_copy(ent_hbm.at[hi_v], h_v, sem)
                cpl = pltpu.async_copy(lab_hbm.at[li_v], l_v, sem)
                cpt = pltpu.async_copy(ent_hbm.at[ti_v], t_v, sem)
                cph.wait()
                cpl.wait()
                cpt.wait()

                def group(g, _):
                    rid = g * L + lid
                    sl = pl.ds(off + g * L, L)
                    shift = jnp.int32(QBITS - 5)
                    msk = jnp.int32(0x60)
                    cb_h = lax.bitwise_and(
                        lax.shift_right_logical(row_v[kh, sl], shift), msk
                    )
                    cb_l = lax.bitwise_and(
                        lax.shift_right_logical(row_v[kl, sl], shift), msk
                    )
                    cb_t = lax.bitwise_and(
                        lax.shift_right_logical(row_v[kt, sl], shift), msk
                    )
                    d = distance(rid, cb_h, cb_l, cb_t)
                    if is_pos:
                        dp_v[sl] = d
                    else:
                        loss = jnp.maximum(
                            jnp.float32(MARGIN) + dp_v[sl] - d, jnp.float32(0.0)
                        )
                        out_v[sl] = loss
                    return _

                lax.fori_loop(0, groups, group, 0)
                return _

            lax.fori_loop(0, nchunks, do_chunk, 0)

        phase(0, 1, 2, True)
        phase(3, 4, 5, False)

        pltpu.sync_copy(out_v, out_hbm.at[0, pl.ds(base, bpw)])

    return sc_kernel


def kernel(positive, negative, embed_entity, embed_label):
    batch = positive.shape[0]
    ent_packed = _pack_table(embed_entity.T)
    lab_packed = _pack_table(embed_label.T)
    sc = _make_sc_kernel(batch)
    return sc(
        positive[:, 0],
        positive[:, 1],
        positive[:, 2],
        negative[:, 0],
        negative[:, 1],
        negative[:, 2],
        ent_packed,
        lab_packed,
    )
